# baseline (speedup 1.0000x reference)
import functools
import math

import jax
import jax.numpy as jnp
from jax import lax
from jax.experimental import pallas as pl
from jax.experimental.pallas import tpu as pltpu
from jax.experimental.pallas import tpu_sc as plsc

_NC = 2
_NS = 16
_LANES = 16
_NW = _NC * _NS

_ROUTE_BLK = 512


def _vreg_gather(v, idx):
    dn = lax.GatherDimensionNumbers(
        offset_dims=(), collapsed_slice_dims=(0,), start_index_map=(0,))
    return lax.gather(v, idx.reshape(_LANES, 1), dn, (1,),
                      mode=lax.GatherScatterMode.PROMISE_IN_BOUNDS)
_DISP_CH = 64
_COMB_CH = 16


def _routing_body(tok_ref, gw_ref, t1_ref, t2_ref, r1_ref, c2_ref,
                  p1_ref, p2_ref, tot_ref, cnt_ref, *, e, cap):
    i = pl.program_id(0)

    @pl.when(i == 0)
    def _init():
        cnt_ref[...] = jnp.zeros_like(cnt_ref)

    x = tok_ref[...]
    gw = gw_ref[...]
    logits = lax.dot_general(x, gw, (((1,), (1,)), ((), ())),
                             preferred_element_type=jnp.float32)
    m = jnp.max(logits, axis=-1, keepdims=True)
    ex = jnp.exp(logits - m)
    probs = ex / jnp.sum(ex, axis=-1, keepdims=True)

    eidx = lax.broadcasted_iota(jnp.int32, probs.shape, 1)
    p1 = jnp.max(probs, axis=-1, keepdims=True)
    is1 = probs == p1
    top1 = jnp.min(jnp.where(is1, eidx, e), axis=-1, keepdims=True)
    mask1 = (eidx == top1)
    probs_ex = jnp.where(mask1, -jnp.inf, probs)
    p2 = jnp.max(probs_ex, axis=-1, keepdims=True)
    is2 = probs_ex == p2
    top2 = jnp.min(jnp.where(is2, eidx, e), axis=-1, keepdims=True)
    mask2 = (eidx == top2)

    blk = probs.shape[0]
    rr = lax.broadcasted_iota(jnp.int32, (blk, blk), 0)
    cc = lax.broadcasted_iota(jnp.int32, (blk, blk), 1)
    tri = (rr >= cc).astype(jnp.float32)
    m1f = mask1.astype(jnp.float32)
    m2f = mask2.astype(jnp.float32)
    cs1 = lax.dot_general(tri, m1f, (((1,), (0,)), ((), ())),
                          preferred_element_type=jnp.float32)
    cs2 = lax.dot_general(tri, m2f, (((1,), (0,)), ((), ())),
                          preferred_element_type=jnp.float32)
    base1 = cnt_ref[0:1, :]
    base2 = cnt_ref[1:2, :]
    r1 = jnp.sum(m1f * (cs1 - 1.0 + base1), axis=-1, keepdims=True)
    c2 = jnp.sum(m2f * (cs2 - 1.0 + base2), axis=-1, keepdims=True)
    cnt_ref[0:1, :] = base1 + jnp.sum(m1f, axis=0, keepdims=True)
    cnt_ref[1:2, :] = base2 + jnp.sum(m2f, axis=0, keepdims=True)

    t1_ref[...] = top1
    t2_ref[...] = top2
    r1_ref[...] = r1.astype(jnp.int32)
    c2_ref[...] = c2.astype(jnp.int32)
    p1_ref[...] = p1
    p2_ref[...] = p2
    tot1 = cnt_ref[0:1, :].astype(jnp.int32)
    pad = jnp.zeros((1, 2 * _LANES - e), jnp.int32)
    tot_ref[...] = jnp.concatenate([tot1, pad], axis=1).reshape(1, 1, 2 * _LANES)


def _routing_call(tokens, gate_weight, cap):
    s, h = tokens.shape
    e = gate_weight.shape[0]
    nblk = s // _ROUTE_BLK
    col = functools.partial(jax.ShapeDtypeStruct, (s, 1))
    out_shapes = (col(jnp.int32), col(jnp.int32), col(jnp.int32),
                  col(jnp.int32), col(jnp.float32), col(jnp.float32),
                  jax.ShapeDtypeStruct((1, 1, 2 * _LANES), jnp.int32))
    col_spec = pl.BlockSpec((_ROUTE_BLK, 1), lambda i: (i, 0))
    return pl.pallas_call(
        functools.partial(_routing_body, e=e, cap=cap),
        grid=(nblk,),
        in_specs=[pl.BlockSpec((_ROUTE_BLK, h), lambda i: (i, 0)),
                  pl.BlockSpec((e, h), lambda i: (0, 0))],
        out_specs=(col_spec, col_spec, col_spec, col_spec, col_spec, col_spec,
                   pl.BlockSpec((1, 1, 2 * _LANES), lambda i: (0, 0, 0))),
        out_shape=out_shapes,
        scratch_shapes=[pltpu.VMEM((2, e), jnp.float32)],
        compiler_params=pltpu.CompilerParams(
            dimension_semantics=("arbitrary",)),
    )(tokens, gate_weight)


def _dispatch_call(t1, t2, r1, c2, p1, p2, tot, tokens, cap, n_disp_rows):
    s, h = tokens.shape
    tpw = s // _NW
    trash = n_disp_rows - _ROW_BLK
    nch = tpw // _DISP_CH
    gpc = _DISP_CH // _LANES

    mesh = plsc.VectorSubcoreMesh(core_axis_name="c", subcore_axis_name="s")

    def body(t1_h, t2_h, r1_h, c2_h, p1_h, p2_h, tot_h, tok_h,
             disp_h, s1_h, s2_h, w1_h, w2_h,
             t1_v, t2_v, r1_v, c2_v, p1_v, p2_v, tot_v,
             s1_v, s2_v, w1_v, w2_v, idx1_v, idx2_v, rows_v, sem):
        wid = lax.axis_index("s") * _NC + lax.axis_index("c")
        base = wid * tpw
        pltpu.sync_copy(t1_h.at[pl.ds(base, tpw)], t1_v)
        pltpu.sync_copy(t2_h.at[pl.ds(base, tpw)], t2_v)
        pltpu.sync_copy(r1_h.at[pl.ds(base, tpw)], r1_v)
        pltpu.sync_copy(c2_h.at[pl.ds(base, tpw)], c2_v)
        pltpu.sync_copy(p1_h.at[pl.ds(base, tpw)], p1_v)
        pltpu.sync_copy(p2_h.at[pl.ds(base, tpw)], p2_v)
        pltpu.sync_copy(tot_h, tot_v)
        tot_vec = tot_v[pl.ds(0, _LANES)]

        for g in range(tpw // _LANES):
            o = g * _LANES
            gt1 = t1_v[pl.ds(o, _LANES)]
            gt2 = t2_v[pl.ds(o, _LANES)]
            gr1 = r1_v[pl.ds(o, _LANES)]
            gc2 = c2_v[pl.ds(o, _LANES)]
            gp1 = p1_v[pl.ds(o, _LANES)]
            gp2 = p2_v[pl.ds(o, _LANES)]
            keep1 = gr1 < cap
            slot1 = gt1 * cap + gr1
            tot2 = _vreg_gather(tot_vec, gt2)
            rank2 = gc2 + tot2
            keep2 = rank2 < cap
            slot2 = gt2 * cap + rank2
            s1_v[pl.ds(o, _LANES)] = jnp.where(keep1, slot1, trash)
            s2_v[pl.ds(o, _LANES)] = jnp.where(keep2, slot2, trash)
            w1_v[pl.ds(o, _LANES)] = jnp.where(keep1, gp1, 0.0)
            w2_v[pl.ds(o, _LANES)] = jnp.where(keep2, gp2, 0.0)
            c, gg = g // gpc, g % gpc
            idx1_v[c, pl.ds(gg * _LANES, _LANES)] = jnp.where(keep1, slot1, trash)
            idx2_v[c, pl.ds(gg * _LANES, _LANES)] = jnp.where(keep2, slot2, trash)

        pltpu.sync_copy(s1_v, s1_h.at[pl.ds(base, tpw)])
        pltpu.sync_copy(s2_v, s2_h.at[pl.ds(base, tpw)])
        pltpu.sync_copy(w1_v, w1_h.at[pl.ds(base, tpw)])
        pltpu.sync_copy(w2_v, w2_h.at[pl.ds(base, tpw)])

        for c in range(nch):
            pltpu.sync_copy(tok_h.at[pl.ds(base + c * _DISP_CH, _DISP_CH)],
                            rows_v)
            d1 = pltpu.async_copy(rows_v, disp_h.at[idx1_v.at[c]], sem)
            d1.wait()
            d2 = pltpu.async_copy(rows_v, disp_h.at[idx2_v.at[c]], sem)
            d2.wait()

    f = pl.kernel(
        body,
        out_type=(jax.ShapeDtypeStruct((n_disp_rows, h), jnp.float32),
                  jax.ShapeDtypeStruct((s,), jnp.int32),
                  jax.ShapeDtypeStruct((s,), jnp.int32),
                  jax.ShapeDtypeStruct((s,), jnp.float32),
                  jax.ShapeDtypeStruct((s,), jnp.float32)),
        mesh=mesh,
        scratch_types=[
            pltpu.VMEM((tpw,), jnp.int32), pltpu.VMEM((tpw,), jnp.int32),
            pltpu.VMEM((tpw,), jnp.int32), pltpu.VMEM((tpw,), jnp.int32),
            pltpu.VMEM((tpw,), jnp.float32), pltpu.VMEM((tpw,), jnp.float32),
            pltpu.VMEM((2 * _LANES,), jnp.int32),
            pltpu.VMEM((tpw,), jnp.int32), pltpu.VMEM((tpw,), jnp.int32),
            pltpu.VMEM((tpw,), jnp.float32), pltpu.VMEM((tpw,), jnp.float32),
            pltpu.VMEM((nch, _DISP_CH), jnp.int32),
            pltpu.VMEM((nch, _DISP_CH), jnp.int32),
            pltpu.VMEM((_DISP_CH, h), jnp.float32),
            pltpu.SemaphoreType.DMA,
        ],
    )
    return f(t1, t2, r1, c2, p1, p2, tot, tokens)


_ROW_BLK = 1280


def _mlp_body(x_ref, wi_ref, wo_ref, o_ref):
    ib = pl.program_id(1)
    xb = x_ref[...].astype(jnp.bfloat16)
    hmid = lax.dot_general(xb, wi_ref[0], (((1,), (0,)), ((), ())),
                           preferred_element_type=jnp.float32)
    g = 0.5 * hmid * (1.0 + lax.erf(hmid * (1.0 / math.sqrt(2.0))))
    contrib = lax.dot_general(g.astype(jnp.bfloat16), wo_ref[0],
                              (((1,), (0,)), ((), ())),
                              preferred_element_type=jnp.float32)

    @pl.when(ib == 0)
    def _set():
        o_ref[...] = contrib

    @pl.when(ib > 0)
    def _acc():
        o_ref[...] += contrib


def _mlp_call(dispatch, wi, wo, cap):
    e, h, inter = wi.shape
    ib_blk = 1024
    nib = inter // ib_blk
    bpe = cap // _ROW_BLK
    return pl.pallas_call(
        _mlp_body,
        grid=(e * bpe + 1, nib),
        in_specs=[pl.BlockSpec((_ROW_BLK, h), lambda rb, ib: (rb, 0)),
                  pl.BlockSpec((1, h, ib_blk),
                               lambda rb, ib: (jnp.minimum(rb // bpe, e - 1),
                                               0, ib)),
                  pl.BlockSpec((1, ib_blk, h),
                               lambda rb, ib: (jnp.minimum(rb // bpe, e - 1),
                                               ib, 0))],
        out_specs=pl.BlockSpec((_ROW_BLK, h), lambda rb, ib: (rb, 0)),
        out_shape=jax.ShapeDtypeStruct((e * cap + _ROW_BLK, h), jnp.float32),
        compiler_params=pltpu.CompilerParams(
            dimension_semantics=("arbitrary", "arbitrary")),
    )(dispatch, wi, wo)


def _combine_call(eo, s1, s2, w1, w2, out_shape):
    s = s1.shape[0]
    h = eo.shape[1]
    tpw = s // _NW
    nch = tpw // _COMB_CH
    ngrp = h // _LANES

    mesh = plsc.VectorSubcoreMesh(core_axis_name="c", subcore_axis_name="s")

    def body(eo_h, s1_h, s2_h, w1_h, w2_h, out_h,
             s1_v, s2_v, w1_v, w2_v,
             r1a_v, r2a_v, r1b_v, r2b_v, acca_v, accb_v,
             sga, sgb, sout):
        wid = lax.axis_index("s") * _NC + lax.axis_index("c")
        base = wid * tpw
        pltpu.sync_copy(s1_h.at[pl.ds(base, tpw)], s1_v)
        pltpu.sync_copy(s2_h.at[pl.ds(base, tpw)], s2_v)
        pltpu.sync_copy(w1_h.at[pl.ds(base, tpw)], w1_v)
        pltpu.sync_copy(w2_h.at[pl.ds(base, tpw)], w2_v)

        def fire(c, r1x, r2x, sg):
            o = c * _COMB_CH
            pltpu.async_copy(eo_h.at[s1_v.at[pl.ds(o, _COMB_CH)]], r1x, sg)
            pltpu.async_copy(eo_h.at[s2_v.at[pl.ds(o, _COMB_CH)]], r2x, sg)

        def drain(r1x, r2x, sg):
            pltpu.make_async_copy(eo_h.at[pl.ds(0, _COMB_CH)], r1x, sg).wait()
            pltpu.make_async_copy(eo_h.at[pl.ds(0, _COMB_CH)], r2x, sg).wait()

        def compute(c, r1x, r2x, accx):
            o = c * _COMB_CH
            for t in range(_COMB_CH):
                tv = t // _LANES, t % _LANES
                wv1 = w1_v[pl.ds(o + tv[0] * _LANES, _LANES)]
                wv2 = w2_v[pl.ds(o + tv[0] * _LANES, _LANES)]
                lane = jnp.full((_LANES,), tv[1], jnp.int32)
                w1t = _vreg_gather(wv1, lane)
                w2t = _vreg_gather(wv2, lane)

                def lanes(l, carry2):
                    row1 = r1x[t, pl.ds(l * _LANES, _LANES)]
                    row2 = r2x[t, pl.ds(l * _LANES, _LANES)]
                    accx[t, pl.ds(l * _LANES, _LANES)] = (w1t * row1
                                                          + w2t * row2)
                    return carry2

                lax.fori_loop(0, ngrp, lanes, 0, unroll=4)
            pltpu.async_copy(accx, out_h.at[pl.ds(base + o, _COMB_CH)], sout)

        def drain_out(accx):
            pltpu.make_async_copy(eo_h.at[pl.ds(0, _COMB_CH)], accx,
                                  sout).wait()

        fire(0, r1a_v, r2a_v, sga)

        def pair(i, carry):
            c = 2 * i
            drain(r1a_v, r2a_v, sga)
            fire(c + 1, r1b_v, r2b_v, sgb)

            @pl.when(i > 0)
            def _():
                drain_out(acca_v)

            compute(c, r1a_v, r2a_v, acca_v)
            drain(r1b_v, r2b_v, sgb)

            @pl.when(c + 2 < nch)
            def _():
                fire(c + 2, r1a_v, r2a_v, sga)

            @pl.when(i > 0)
            def _():
                drain_out(accb_v)

            compute(c + 1, r1b_v, r2b_v, accb_v)
            return carry

        lax.fori_loop(0, nch // 2, pair, 0)
        drain_out(acca_v)
        drain_out(accb_v)

    f = pl.kernel(
        body,
        out_type=jax.ShapeDtypeStruct((s, h), jnp.float32),
        mesh=mesh,
        scratch_types=[
            pltpu.VMEM((tpw,), jnp.int32), pltpu.VMEM((tpw,), jnp.int32),
            pltpu.VMEM((tpw,), jnp.float32), pltpu.VMEM((tpw,), jnp.float32),
            pltpu.VMEM((_COMB_CH, h), jnp.float32),
            pltpu.VMEM((_COMB_CH, h), jnp.float32),
            pltpu.VMEM((_COMB_CH, h), jnp.float32),
            pltpu.VMEM((_COMB_CH, h), jnp.float32),
            pltpu.VMEM((_COMB_CH, h), jnp.float32),
            pltpu.VMEM((_COMB_CH, h), jnp.float32),
            pltpu.SemaphoreType.DMA, pltpu.SemaphoreType.DMA,
            pltpu.SemaphoreType.DMA,
        ],
    )
    return f(eo, s1, s2, w1, w2)


def kernel(inputs, gate_weight, wi, wo):
    orig_shape = inputs.shape
    h = inputs.shape[-1]
    tokens = inputs.reshape(-1, h)
    s = tokens.shape[0]
    e = gate_weight.shape[0]
    cap = math.floor(2 * 1.25 * s / e)
    cap += cap % 2
    cap = max(cap, 4)
    n_disp_rows = e * cap + _ROW_BLK

    t1, t2, r1, c2, p1, p2, tot = _routing_call(tokens, gate_weight, cap)
    t1, t2, r1, c2, p1, p2 = (a.reshape(s) for a in (t1, t2, r1, c2, p1, p2))
    tot = tot.reshape(2 * _LANES)

    disp, s1, s2, w1, w2 = _dispatch_call(
        t1, t2, r1, c2, p1, p2, tot, tokens, cap, n_disp_rows)
    eo = _mlp_call(disp, wi.astype(jnp.bfloat16), wo.astype(jnp.bfloat16), cap)
    out = _combine_call(eo, s1, s2, w1, w2, (s, h))
    return out.reshape(orig_shape)


# speedup vs baseline: 1.2291x; 1.2291x over previous
import functools
import math

import jax
import jax.numpy as jnp
from jax import lax
from jax.experimental import pallas as pl
from jax.experimental.pallas import tpu as pltpu
from jax.experimental.pallas import tpu_sc as plsc

_NC = 2
_NS = 16
_LANES = 16
_NW = _NC * _NS

_ROUTE_BLK = 512


def _vreg_gather(v, idx):
    dn = lax.GatherDimensionNumbers(
        offset_dims=(), collapsed_slice_dims=(0,), start_index_map=(0,))
    return lax.gather(v, idx.reshape(_LANES, 1), dn, (1,),
                      mode=lax.GatherScatterMode.PROMISE_IN_BOUNDS)
_DISP_CH = 32
_COMB_CH = 16


def _routing_body(tok_ref, gw_ref, t1_ref, t2_ref, r1_ref, c2_ref,
                  p1_ref, p2_ref, tot_ref, cnt_ref, *, e, cap):
    i = pl.program_id(0)

    @pl.when(i == 0)
    def _init():
        cnt_ref[...] = jnp.zeros_like(cnt_ref)

    x = tok_ref[...]
    gw = gw_ref[...]
    logits = lax.dot_general(x, gw, (((1,), (1,)), ((), ())),
                             preferred_element_type=jnp.float32)
    m = jnp.max(logits, axis=-1, keepdims=True)
    ex = jnp.exp(logits - m)
    probs = ex / jnp.sum(ex, axis=-1, keepdims=True)

    eidx = lax.broadcasted_iota(jnp.int32, probs.shape, 1)
    p1 = jnp.max(probs, axis=-1, keepdims=True)
    is1 = probs == p1
    top1 = jnp.min(jnp.where(is1, eidx, e), axis=-1, keepdims=True)
    mask1 = (eidx == top1)
    probs_ex = jnp.where(mask1, -jnp.inf, probs)
    p2 = jnp.max(probs_ex, axis=-1, keepdims=True)
    is2 = probs_ex == p2
    top2 = jnp.min(jnp.where(is2, eidx, e), axis=-1, keepdims=True)
    mask2 = (eidx == top2)

    blk = probs.shape[0]
    rr = lax.broadcasted_iota(jnp.int32, (blk, blk), 0)
    cc = lax.broadcasted_iota(jnp.int32, (blk, blk), 1)
    tri = (rr >= cc).astype(jnp.float32)
    m1f = mask1.astype(jnp.float32)
    m2f = mask2.astype(jnp.float32)
    cs1 = lax.dot_general(tri, m1f, (((1,), (0,)), ((), ())),
                          preferred_element_type=jnp.float32)
    cs2 = lax.dot_general(tri, m2f, (((1,), (0,)), ((), ())),
                          preferred_element_type=jnp.float32)
    base1 = cnt_ref[0:1, :]
    base2 = cnt_ref[1:2, :]
    r1 = jnp.sum(m1f * (cs1 - 1.0 + base1), axis=-1, keepdims=True)
    c2 = jnp.sum(m2f * (cs2 - 1.0 + base2), axis=-1, keepdims=True)
    cnt_ref[0:1, :] = base1 + jnp.sum(m1f, axis=0, keepdims=True)
    cnt_ref[1:2, :] = base2 + jnp.sum(m2f, axis=0, keepdims=True)

    t1_ref[...] = top1
    t2_ref[...] = top2
    r1_ref[...] = r1.astype(jnp.int32)
    c2_ref[...] = c2.astype(jnp.int32)
    p1_ref[...] = p1
    p2_ref[...] = p2
    tot1 = cnt_ref[0:1, :].astype(jnp.int32)
    pad = jnp.zeros((1, 2 * _LANES - e), jnp.int32)
    tot_ref[...] = jnp.concatenate([tot1, pad], axis=1).reshape(1, 1, 2 * _LANES)


def _routing_call(tokens, gate_weight, cap):
    s, h = tokens.shape
    e = gate_weight.shape[0]
    nblk = s // _ROUTE_BLK
    col = functools.partial(jax.ShapeDtypeStruct, (s, 1))
    out_shapes = (col(jnp.int32), col(jnp.int32), col(jnp.int32),
                  col(jnp.int32), col(jnp.float32), col(jnp.float32),
                  jax.ShapeDtypeStruct((1, 1, 2 * _LANES), jnp.int32))
    col_spec = pl.BlockSpec((_ROUTE_BLK, 1), lambda i: (i, 0))
    return pl.pallas_call(
        functools.partial(_routing_body, e=e, cap=cap),
        grid=(nblk,),
        in_specs=[pl.BlockSpec((_ROUTE_BLK, h), lambda i: (i, 0)),
                  pl.BlockSpec((e, h), lambda i: (0, 0))],
        out_specs=(col_spec, col_spec, col_spec, col_spec, col_spec, col_spec,
                   pl.BlockSpec((1, 1, 2 * _LANES), lambda i: (0, 0, 0))),
        out_shape=out_shapes,
        scratch_shapes=[pltpu.VMEM((2, e), jnp.float32)],
        compiler_params=pltpu.CompilerParams(
            dimension_semantics=("arbitrary",)),
    )(tokens, gate_weight)


def _dispatch_call(t1, t2, r1, c2, p1, p2, tot, tokens, cap, n_disp_rows):
    s, h = tokens.shape
    tpw = s // _NW
    trash = n_disp_rows - _ROW_BLK
    nch = tpw // _DISP_CH
    gpc = _DISP_CH // _LANES

    mesh = plsc.VectorSubcoreMesh(core_axis_name="c", subcore_axis_name="s")

    def body(t1_h, t2_h, r1_h, c2_h, p1_h, p2_h, tot_h, tok_h,
             disp_h, s1_h, s2_h, w1_h, w2_h,
             t1_v, t2_v, r1_v, c2_v, p1_v, p2_v, tot_v,
             s1_v, s2_v, w1_v, w2_v, idx1_v, idx2_v,
             rowsa_v, rowsb_v, sld, sst):
        wid = lax.axis_index("s") * _NC + lax.axis_index("c")
        base = wid * tpw
        pltpu.sync_copy(t1_h.at[pl.ds(base, tpw)], t1_v)
        pltpu.sync_copy(t2_h.at[pl.ds(base, tpw)], t2_v)
        pltpu.sync_copy(r1_h.at[pl.ds(base, tpw)], r1_v)
        pltpu.sync_copy(c2_h.at[pl.ds(base, tpw)], c2_v)
        pltpu.sync_copy(p1_h.at[pl.ds(base, tpw)], p1_v)
        pltpu.sync_copy(p2_h.at[pl.ds(base, tpw)], p2_v)
        pltpu.sync_copy(tot_h, tot_v)
        tot_vec = tot_v[pl.ds(0, _LANES)]

        for g in range(tpw // _LANES):
            o = g * _LANES
            gt1 = t1_v[pl.ds(o, _LANES)]
            gt2 = t2_v[pl.ds(o, _LANES)]
            gr1 = r1_v[pl.ds(o, _LANES)]
            gc2 = c2_v[pl.ds(o, _LANES)]
            gp1 = p1_v[pl.ds(o, _LANES)]
            gp2 = p2_v[pl.ds(o, _LANES)]
            keep1 = gr1 < cap
            slot1 = gt1 * cap + gr1
            tot2 = _vreg_gather(tot_vec, gt2)
            rank2 = gc2 + tot2
            keep2 = rank2 < cap
            slot2 = gt2 * cap + rank2
            s1_v[pl.ds(o, _LANES)] = jnp.where(keep1, slot1, trash)
            s2_v[pl.ds(o, _LANES)] = jnp.where(keep2, slot2, trash)
            w1_v[pl.ds(o, _LANES)] = jnp.where(keep1, gp1, 0.0)
            w2_v[pl.ds(o, _LANES)] = jnp.where(keep2, gp2, 0.0)
            c, gg = g // gpc, g % gpc
            idx1_v[c, pl.ds(gg * _LANES, _LANES)] = jnp.where(keep1, slot1, trash)
            idx2_v[c, pl.ds(gg * _LANES, _LANES)] = jnp.where(keep2, slot2, trash)

        pltpu.sync_copy(s1_v, s1_h.at[pl.ds(base, tpw)])
        pltpu.sync_copy(s2_v, s2_h.at[pl.ds(base, tpw)])
        pltpu.sync_copy(w1_v, w1_h.at[pl.ds(base, tpw)])
        pltpu.sync_copy(w2_v, w2_h.at[pl.ds(base, tpw)])

        bufs = (rowsa_v, rowsb_v)
        pltpu.async_copy(tok_h.at[pl.ds(base, _DISP_CH)], rowsa_v, sld)
        for c in range(nch):
            cur = bufs[c % 2]
            pltpu.make_async_copy(tok_h.at[pl.ds(0, _DISP_CH)], cur,
                                  sld).wait()
            if c + 1 < nch:
                pltpu.async_copy(
                    tok_h.at[pl.ds(base + (c + 1) * _DISP_CH, _DISP_CH)],
                    bufs[(c + 1) % 2], sld)
            d1 = pltpu.async_copy(cur, disp_h.at[idx1_v.at[c]], sst)
            d2 = pltpu.async_copy(cur, disp_h.at[idx2_v.at[c]], sst)
            d1.wait()
            d2.wait()

    f = pl.kernel(
        body,
        out_type=(jax.ShapeDtypeStruct((n_disp_rows, h), jnp.float32),
                  jax.ShapeDtypeStruct((s,), jnp.int32),
                  jax.ShapeDtypeStruct((s,), jnp.int32),
                  jax.ShapeDtypeStruct((s,), jnp.float32),
                  jax.ShapeDtypeStruct((s,), jnp.float32)),
        mesh=mesh,
        scratch_types=[
            pltpu.VMEM((tpw,), jnp.int32), pltpu.VMEM((tpw,), jnp.int32),
            pltpu.VMEM((tpw,), jnp.int32), pltpu.VMEM((tpw,), jnp.int32),
            pltpu.VMEM((tpw,), jnp.float32), pltpu.VMEM((tpw,), jnp.float32),
            pltpu.VMEM((2 * _LANES,), jnp.int32),
            pltpu.VMEM((tpw,), jnp.int32), pltpu.VMEM((tpw,), jnp.int32),
            pltpu.VMEM((tpw,), jnp.float32), pltpu.VMEM((tpw,), jnp.float32),
            pltpu.VMEM((nch, _DISP_CH), jnp.int32),
            pltpu.VMEM((nch, _DISP_CH), jnp.int32),
            pltpu.VMEM((_DISP_CH, h), jnp.float32),
            pltpu.VMEM((_DISP_CH, h), jnp.float32),
            pltpu.SemaphoreType.DMA, pltpu.SemaphoreType.DMA,
        ],
    )
    return f(t1, t2, r1, c2, p1, p2, tot, tokens)


_ROW_BLK = 1280


def _mlp_body(x_ref, wi_ref, wo_ref, o_ref, *, nrb_real):
    rb = pl.program_id(0)
    ib = pl.program_id(1)

    @pl.when((rb < nrb_real) | (ib == 0))
    def _compute():
        hmid = lax.dot_general(x_ref[...], wi_ref[0], (((1,), (0,)), ((), ())),
                               preferred_element_type=jnp.float32)
        g = 0.5 * hmid * (1.0 + lax.erf(hmid * (1.0 / math.sqrt(2.0))))
        contrib = lax.dot_general(g, wo_ref[0], (((1,), (0,)), ((), ())),
                                  preferred_element_type=jnp.float32)

        @pl.when(ib == 0)
        def _set():
            o_ref[...] = contrib

        @pl.when(ib > 0)
        def _acc():
            o_ref[...] += contrib


def _mlp_call(dispatch, wi, wo, cap):
    e, h, inter = wi.shape
    ib_blk = 1024
    nib = inter // ib_blk
    bpe = cap // _ROW_BLK
    return pl.pallas_call(
        functools.partial(_mlp_body, nrb_real=e * bpe),
        grid=(e * bpe + 1, nib),
        in_specs=[pl.BlockSpec((_ROW_BLK, h), lambda rb, ib: (rb, 0)),
                  pl.BlockSpec((1, h, ib_blk),
                               lambda rb, ib: (jnp.minimum(rb // bpe, e - 1),
                                               0, ib)),
                  pl.BlockSpec((1, ib_blk, h),
                               lambda rb, ib: (jnp.minimum(rb // bpe, e - 1),
                                               ib, 0))],
        out_specs=pl.BlockSpec((_ROW_BLK, h), lambda rb, ib: (rb, 0)),
        out_shape=jax.ShapeDtypeStruct((e * cap + _ROW_BLK, h), jnp.float32),
        compiler_params=pltpu.CompilerParams(
            dimension_semantics=("arbitrary", "arbitrary")),
    )(dispatch, wi, wo)


def _combine_call(eo, s1, s2, w1, w2, out_shape):
    s = s1.shape[0]
    h = eo.shape[1]
    tpw = s // _NW
    nch = tpw // _COMB_CH
    ngrp = h // _LANES

    mesh = plsc.VectorSubcoreMesh(core_axis_name="c", subcore_axis_name="s")

    def body(eo_h, s1_h, s2_h, w1_h, w2_h, out_h,
             s1_v, s2_v, w1_v, w2_v,
             r1a_v, r2a_v, r1b_v, r2b_v, acca_v, accb_v,
             sga, sgb, sout):
        wid = lax.axis_index("s") * _NC + lax.axis_index("c")
        base = wid * tpw
        pltpu.sync_copy(s1_h.at[pl.ds(base, tpw)], s1_v)
        pltpu.sync_copy(s2_h.at[pl.ds(base, tpw)], s2_v)
        pltpu.sync_copy(w1_h.at[pl.ds(base, tpw)], w1_v)
        pltpu.sync_copy(w2_h.at[pl.ds(base, tpw)], w2_v)

        def fire(c, r1x, r2x, sg):
            o = c * _COMB_CH
            pltpu.async_copy(eo_h.at[s1_v.at[pl.ds(o, _COMB_CH)]], r1x, sg)
            pltpu.async_copy(eo_h.at[s2_v.at[pl.ds(o, _COMB_CH)]], r2x, sg)

        def drain(r1x, r2x, sg):
            pltpu.make_async_copy(eo_h.at[pl.ds(0, _COMB_CH)], r1x, sg).wait()
            pltpu.make_async_copy(eo_h.at[pl.ds(0, _COMB_CH)], r2x, sg).wait()

        def compute(c, r1x, r2x, accx):
            o = c * _COMB_CH
            for t in range(_COMB_CH):
                tv = t // _LANES, t % _LANES
                wv1 = w1_v[pl.ds(o + tv[0] * _LANES, _LANES)]
                wv2 = w2_v[pl.ds(o + tv[0] * _LANES, _LANES)]
                lane = jnp.full((_LANES,), tv[1], jnp.int32)
                w1t = _vreg_gather(wv1, lane)
                w2t = _vreg_gather(wv2, lane)

                def lanes(l, carry2):
                    row1 = r1x[t, pl.ds(l * _LANES, _LANES)]
                    row2 = r2x[t, pl.ds(l * _LANES, _LANES)]
                    accx[t, pl.ds(l * _LANES, _LANES)] = (w1t * row1
                                                          + w2t * row2)
                    return carry2

                lax.fori_loop(0, ngrp, lanes, 0, unroll=4)
            pltpu.async_copy(accx, out_h.at[pl.ds(base + o, _COMB_CH)], sout)

        def drain_out(accx):
            pltpu.make_async_copy(eo_h.at[pl.ds(0, _COMB_CH)], accx,
                                  sout).wait()

        fire(0, r1a_v, r2a_v, sga)

        def pair(i, carry):
            c = 2 * i
            drain(r1a_v, r2a_v, sga)
            fire(c + 1, r1b_v, r2b_v, sgb)

            @pl.when(i > 0)
            def _():
                drain_out(acca_v)

            compute(c, r1a_v, r2a_v, acca_v)
            drain(r1b_v, r2b_v, sgb)

            @pl.when(c + 2 < nch)
            def _():
                fire(c + 2, r1a_v, r2a_v, sga)

            @pl.when(i > 0)
            def _():
                drain_out(accb_v)

            compute(c + 1, r1b_v, r2b_v, accb_v)
            return carry

        lax.fori_loop(0, nch // 2, pair, 0)
        drain_out(acca_v)
        drain_out(accb_v)

    f = pl.kernel(
        body,
        out_type=jax.ShapeDtypeStruct((s, h), jnp.float32),
        mesh=mesh,
        scratch_types=[
            pltpu.VMEM((tpw,), jnp.int32), pltpu.VMEM((tpw,), jnp.int32),
            pltpu.VMEM((tpw,), jnp.float32), pltpu.VMEM((tpw,), jnp.float32),
            pltpu.VMEM((_COMB_CH, h), jnp.float32),
            pltpu.VMEM((_COMB_CH, h), jnp.float32),
            pltpu.VMEM((_COMB_CH, h), jnp.float32),
            pltpu.VMEM((_COMB_CH, h), jnp.float32),
            pltpu.VMEM((_COMB_CH, h), jnp.float32),
            pltpu.VMEM((_COMB_CH, h), jnp.float32),
            pltpu.SemaphoreType.DMA, pltpu.SemaphoreType.DMA,
            pltpu.SemaphoreType.DMA,
        ],
    )
    return f(eo, s1, s2, w1, w2)


def kernel(inputs, gate_weight, wi, wo):
    orig_shape = inputs.shape
    h = inputs.shape[-1]
    tokens = inputs.reshape(-1, h)
    s = tokens.shape[0]
    e = gate_weight.shape[0]
    cap = math.floor(2 * 1.25 * s / e)
    cap += cap % 2
    cap = max(cap, 4)
    n_disp_rows = e * cap + _ROW_BLK

    t1, t2, r1, c2, p1, p2, tot = _routing_call(tokens, gate_weight, cap)
    t1, t2, r1, c2, p1, p2 = (a.reshape(s) for a in (t1, t2, r1, c2, p1, p2))
    tot = tot.reshape(2 * _LANES)

    disp, s1, s2, w1, w2 = _dispatch_call(
        t1, t2, r1, c2, p1, p2, tot, tokens, cap, n_disp_rows)
    eo = _mlp_call(disp, wi, wo, cap)
    out = _combine_call(eo, s1, s2, w1, w2, (s, h))
    return out.reshape(orig_shape)


# speedup vs baseline: 1.2305x; 1.0011x over previous
import functools
import math

import jax
import jax.numpy as jnp
from jax import lax
from jax.experimental import pallas as pl
from jax.experimental.pallas import tpu as pltpu
from jax.experimental.pallas import tpu_sc as plsc

_NC = 2
_NS = 16
_LANES = 16
_NW = _NC * _NS

_ROUTE_BLK = 1024


def _vreg_gather(v, idx):
    dn = lax.GatherDimensionNumbers(
        offset_dims=(), collapsed_slice_dims=(0,), start_index_map=(0,))
    return lax.gather(v, idx.reshape(_LANES, 1), dn, (1,),
                      mode=lax.GatherScatterMode.PROMISE_IN_BOUNDS)
_DISP_CH = 32
_COMB_CH = 16


def _routing_body(tok_ref, gw_ref, t1_ref, t2_ref, r1_ref, c2_ref,
                  p1_ref, p2_ref, tot_ref, cnt_ref, *, e, cap):
    i = pl.program_id(0)

    @pl.when(i == 0)
    def _init():
        cnt_ref[...] = jnp.zeros_like(cnt_ref)

    x = tok_ref[...]
    gw = gw_ref[...]
    logits = lax.dot_general(x, gw, (((1,), (1,)), ((), ())),
                             preferred_element_type=jnp.float32)
    m = jnp.max(logits, axis=-1, keepdims=True)
    ex = jnp.exp(logits - m)
    probs = ex / jnp.sum(ex, axis=-1, keepdims=True)

    eidx = lax.broadcasted_iota(jnp.int32, probs.shape, 1)
    p1 = jnp.max(probs, axis=-1, keepdims=True)
    is1 = probs == p1
    top1 = jnp.min(jnp.where(is1, eidx, e), axis=-1, keepdims=True)
    mask1 = (eidx == top1)
    probs_ex = jnp.where(mask1, -jnp.inf, probs)
    p2 = jnp.max(probs_ex, axis=-1, keepdims=True)
    is2 = probs_ex == p2
    top2 = jnp.min(jnp.where(is2, eidx, e), axis=-1, keepdims=True)
    mask2 = (eidx == top2)

    blk = probs.shape[0]
    rr = lax.broadcasted_iota(jnp.int32, (blk, blk), 0)
    cc = lax.broadcasted_iota(jnp.int32, (blk, blk), 1)
    tri = (rr >= cc).astype(jnp.float32)
    m1f = mask1.astype(jnp.float32)
    m2f = mask2.astype(jnp.float32)
    cs1 = lax.dot_general(tri, m1f, (((1,), (0,)), ((), ())),
                          preferred_element_type=jnp.float32)
    cs2 = lax.dot_general(tri, m2f, (((1,), (0,)), ((), ())),
                          preferred_element_type=jnp.float32)
    base1 = cnt_ref[0:1, :]
    base2 = cnt_ref[1:2, :]
    r1 = jnp.sum(m1f * (cs1 - 1.0 + base1), axis=-1, keepdims=True)
    c2 = jnp.sum(m2f * (cs2 - 1.0 + base2), axis=-1, keepdims=True)
    cnt_ref[0:1, :] = base1 + jnp.sum(m1f, axis=0, keepdims=True)
    cnt_ref[1:2, :] = base2 + jnp.sum(m2f, axis=0, keepdims=True)

    t1_ref[...] = top1
    t2_ref[...] = top2
    r1_ref[...] = r1.astype(jnp.int32)
    c2_ref[...] = c2.astype(jnp.int32)
    p1_ref[...] = p1
    p2_ref[...] = p2
    tot1 = cnt_ref[0:1, :].astype(jnp.int32)
    pad = jnp.zeros((1, 2 * _LANES - e), jnp.int32)
    tot_ref[...] = jnp.concatenate([tot1, pad], axis=1).reshape(1, 1, 2 * _LANES)


def _routing_call(tokens, gate_weight, cap):
    s, h = tokens.shape
    e = gate_weight.shape[0]
    nblk = s // _ROUTE_BLK
    col = functools.partial(jax.ShapeDtypeStruct, (s, 1))
    out_shapes = (col(jnp.int32), col(jnp.int32), col(jnp.int32),
                  col(jnp.int32), col(jnp.float32), col(jnp.float32),
                  jax.ShapeDtypeStruct((1, 1, 2 * _LANES), jnp.int32))
    col_spec = pl.BlockSpec((_ROUTE_BLK, 1), lambda i: (i, 0))
    return pl.pallas_call(
        functools.partial(_routing_body, e=e, cap=cap),
        grid=(nblk,),
        in_specs=[pl.BlockSpec((_ROUTE_BLK, h), lambda i: (i, 0)),
                  pl.BlockSpec((e, h), lambda i: (0, 0))],
        out_specs=(col_spec, col_spec, col_spec, col_spec, col_spec, col_spec,
                   pl.BlockSpec((1, 1, 2 * _LANES), lambda i: (0, 0, 0))),
        out_shape=out_shapes,
        scratch_shapes=[pltpu.VMEM((2, e), jnp.float32)],
        compiler_params=pltpu.CompilerParams(
            dimension_semantics=("arbitrary",)),
    )(tokens, gate_weight)


def _dispatch_call(t1, t2, r1, c2, p1, p2, tot, tokens, cap, n_disp_rows):
    s, h = tokens.shape
    tpw = s // _NW
    trash = n_disp_rows - _ROW_BLK
    nch = tpw // _DISP_CH
    gpc = _DISP_CH // _LANES

    mesh = plsc.VectorSubcoreMesh(core_axis_name="c", subcore_axis_name="s")

    def body(t1_h, t2_h, r1_h, c2_h, p1_h, p2_h, tot_h, tok_h,
             disp_h, s1_h, s2_h, w1_h, w2_h,
             t1_v, t2_v, r1_v, c2_v, p1_v, p2_v, tot_v,
             s1_v, s2_v, w1_v, w2_v, idx1_v, idx2_v,
             rowsa_v, rowsb_v, sld, sst):
        wid = lax.axis_index("s") * _NC + lax.axis_index("c")
        base = wid * tpw
        pltpu.sync_copy(t1_h.at[pl.ds(base, tpw)], t1_v)
        pltpu.sync_copy(t2_h.at[pl.ds(base, tpw)], t2_v)
        pltpu.sync_copy(r1_h.at[pl.ds(base, tpw)], r1_v)
        pltpu.sync_copy(c2_h.at[pl.ds(base, tpw)], c2_v)
        pltpu.sync_copy(p1_h.at[pl.ds(base, tpw)], p1_v)
        pltpu.sync_copy(p2_h.at[pl.ds(base, tpw)], p2_v)
        pltpu.sync_copy(tot_h, tot_v)
        tot_vec = tot_v[pl.ds(0, _LANES)]

        for g in range(tpw // _LANES):
            o = g * _LANES
            gt1 = t1_v[pl.ds(o, _LANES)]
            gt2 = t2_v[pl.ds(o, _LANES)]
            gr1 = r1_v[pl.ds(o, _LANES)]
            gc2 = c2_v[pl.ds(o, _LANES)]
            gp1 = p1_v[pl.ds(o, _LANES)]
            gp2 = p2_v[pl.ds(o, _LANES)]
            keep1 = gr1 < cap
            slot1 = gt1 * cap + gr1
            tot2 = _vreg_gather(tot_vec, gt2)
            rank2 = gc2 + tot2
            keep2 = rank2 < cap
            slot2 = gt2 * cap + rank2
            s1_v[pl.ds(o, _LANES)] = jnp.where(keep1, slot1, trash)
            s2_v[pl.ds(o, _LANES)] = jnp.where(keep2, slot2, trash)
            w1_v[pl.ds(o, _LANES)] = jnp.where(keep1, gp1, 0.0)
            w2_v[pl.ds(o, _LANES)] = jnp.where(keep2, gp2, 0.0)
            c, gg = g // gpc, g % gpc
            idx1_v[c, pl.ds(gg * _LANES, _LANES)] = jnp.where(keep1, slot1, trash)
            idx2_v[c, pl.ds(gg * _LANES, _LANES)] = jnp.where(keep2, slot2, trash)

        pltpu.sync_copy(s1_v, s1_h.at[pl.ds(base, tpw)])
        pltpu.sync_copy(s2_v, s2_h.at[pl.ds(base, tpw)])
        pltpu.sync_copy(w1_v, w1_h.at[pl.ds(base, tpw)])
        pltpu.sync_copy(w2_v, w2_h.at[pl.ds(base, tpw)])

        bufs = (rowsa_v, rowsb_v)
        pltpu.async_copy(tok_h.at[pl.ds(base, _DISP_CH)], rowsa_v, sld)
        for c in range(nch):
            cur = bufs[c % 2]
            pltpu.make_async_copy(tok_h.at[pl.ds(0, _DISP_CH)], cur,
                                  sld).wait()
            if c + 1 < nch:
                pltpu.async_copy(
                    tok_h.at[pl.ds(base + (c + 1) * _DISP_CH, _DISP_CH)],
                    bufs[(c + 1) % 2], sld)
            d1 = pltpu.async_copy(cur, disp_h.at[idx1_v.at[c]], sst)
            d2 = pltpu.async_copy(cur, disp_h.at[idx2_v.at[c]], sst)
            d1.wait()
            d2.wait()

    f = pl.kernel(
        body,
        out_type=(jax.ShapeDtypeStruct((n_disp_rows, h), jnp.float32),
                  jax.ShapeDtypeStruct((s,), jnp.int32),
                  jax.ShapeDtypeStruct((s,), jnp.int32),
                  jax.ShapeDtypeStruct((s,), jnp.float32),
                  jax.ShapeDtypeStruct((s,), jnp.float32)),
        mesh=mesh,
        scratch_types=[
            pltpu.VMEM((tpw,), jnp.int32), pltpu.VMEM((tpw,), jnp.int32),
            pltpu.VMEM((tpw,), jnp.int32), pltpu.VMEM((tpw,), jnp.int32),
            pltpu.VMEM((tpw,), jnp.float32), pltpu.VMEM((tpw,), jnp.float32),
            pltpu.VMEM((2 * _LANES,), jnp.int32),
            pltpu.VMEM((tpw,), jnp.int32), pltpu.VMEM((tpw,), jnp.int32),
            pltpu.VMEM((tpw,), jnp.float32), pltpu.VMEM((tpw,), jnp.float32),
            pltpu.VMEM((nch, _DISP_CH), jnp.int32),
            pltpu.VMEM((nch, _DISP_CH), jnp.int32),
            pltpu.VMEM((_DISP_CH, h), jnp.float32),
            pltpu.VMEM((_DISP_CH, h), jnp.float32),
            pltpu.SemaphoreType.DMA, pltpu.SemaphoreType.DMA,
        ],
    )
    return f(t1, t2, r1, c2, p1, p2, tot, tokens)


_ROW_BLK = 1280


def _mlp_body(x_ref, wi_ref, wo_ref, o_ref, *, nrb_real):
    rb = pl.program_id(0)
    ib = pl.program_id(1)

    @pl.when((rb < nrb_real) | (ib == 0))
    def _compute():
        hmid = lax.dot_general(x_ref[...], wi_ref[0], (((1,), (0,)), ((), ())),
                               preferred_element_type=jnp.float32)
        g = 0.5 * hmid * (1.0 + lax.erf(hmid * (1.0 / math.sqrt(2.0))))
        contrib = lax.dot_general(g, wo_ref[0], (((1,), (0,)), ((), ())),
                                  preferred_element_type=jnp.float32)

        @pl.when(ib == 0)
        def _set():
            o_ref[...] = contrib

        @pl.when(ib > 0)
        def _acc():
            o_ref[...] += contrib


def _mlp_call(dispatch, wi, wo, cap):
    e, h, inter = wi.shape
    ib_blk = 1024
    nib = inter // ib_blk
    bpe = cap // _ROW_BLK

    def _ibs(rb, ib):
        return jnp.where(rb % 2 == 0, ib, nib - 1 - ib)

    return pl.pallas_call(
        functools.partial(_mlp_body, nrb_real=e * bpe),
        grid=(e * bpe + 1, nib),
        in_specs=[pl.BlockSpec((_ROW_BLK, h), lambda rb, ib: (rb, 0)),
                  pl.BlockSpec((1, h, ib_blk),
                               lambda rb, ib: (jnp.minimum(rb // bpe, e - 1),
                                               0, _ibs(rb, ib))),
                  pl.BlockSpec((1, ib_blk, h),
                               lambda rb, ib: (jnp.minimum(rb // bpe, e - 1),
                                               _ibs(rb, ib), 0))],
        out_specs=pl.BlockSpec((_ROW_BLK, h), lambda rb, ib: (rb, 0)),
        out_shape=jax.ShapeDtypeStruct((e * cap + _ROW_BLK, h), jnp.float32),
        compiler_params=pltpu.CompilerParams(
            dimension_semantics=("arbitrary", "arbitrary")),
    )(dispatch, wi, wo)


def _combine_call(eo, s1, s2, w1, w2, out_shape):
    s = s1.shape[0]
    h = eo.shape[1]
    tpw = s // _NW
    nch = tpw // _COMB_CH
    ngrp = h // _LANES

    mesh = plsc.VectorSubcoreMesh(core_axis_name="c", subcore_axis_name="s")

    def body(eo_h, s1_h, s2_h, w1_h, w2_h, out_h,
             s1_v, s2_v, w1_v, w2_v,
             r1a_v, r2a_v, r1b_v, r2b_v, acca_v, accb_v,
             sga, sgb, sout):
        wid = lax.axis_index("s") * _NC + lax.axis_index("c")
        base = wid * tpw
        pltpu.sync_copy(s1_h.at[pl.ds(base, tpw)], s1_v)
        pltpu.sync_copy(s2_h.at[pl.ds(base, tpw)], s2_v)
        pltpu.sync_copy(w1_h.at[pl.ds(base, tpw)], w1_v)
        pltpu.sync_copy(w2_h.at[pl.ds(base, tpw)], w2_v)

        def fire(c, r1x, r2x, sg):
            o = c * _COMB_CH
            pltpu.async_copy(eo_h.at[s1_v.at[pl.ds(o, _COMB_CH)]], r1x, sg)
            pltpu.async_copy(eo_h.at[s2_v.at[pl.ds(o, _COMB_CH)]], r2x, sg)

        def drain(r1x, r2x, sg):
            pltpu.make_async_copy(eo_h.at[pl.ds(0, _COMB_CH)], r1x, sg).wait()
            pltpu.make_async_copy(eo_h.at[pl.ds(0, _COMB_CH)], r2x, sg).wait()

        def compute(c, r1x, r2x, accx):
            o = c * _COMB_CH
            for t in range(_COMB_CH):
                tv = t // _LANES, t % _LANES
                wv1 = w1_v[pl.ds(o + tv[0] * _LANES, _LANES)]
                wv2 = w2_v[pl.ds(o + tv[0] * _LANES, _LANES)]
                lane = jnp.full((_LANES,), tv[1], jnp.int32)
                w1t = _vreg_gather(wv1, lane)
                w2t = _vreg_gather(wv2, lane)

                def lanes(l, carry2):
                    row1 = r1x[t, pl.ds(l * _LANES, _LANES)]
                    row2 = r2x[t, pl.ds(l * _LANES, _LANES)]
                    accx[t, pl.ds(l * _LANES, _LANES)] = (w1t * row1
                                                          + w2t * row2)
                    return carry2

                lax.fori_loop(0, ngrp, lanes, 0, unroll=4)
            pltpu.async_copy(accx, out_h.at[pl.ds(base + o, _COMB_CH)], sout)

        def drain_out(accx):
            pltpu.make_async_copy(eo_h.at[pl.ds(0, _COMB_CH)], accx,
                                  sout).wait()

        fire(0, r1a_v, r2a_v, sga)

        def pair(i, carry):
            c = 2 * i
            drain(r1a_v, r2a_v, sga)
            fire(c + 1, r1b_v, r2b_v, sgb)

            @pl.when(i > 0)
            def _():
                drain_out(acca_v)

            compute(c, r1a_v, r2a_v, acca_v)
            drain(r1b_v, r2b_v, sgb)

            @pl.when(c + 2 < nch)
            def _():
                fire(c + 2, r1a_v, r2a_v, sga)

            @pl.when(i > 0)
            def _():
                drain_out(accb_v)

            compute(c + 1, r1b_v, r2b_v, accb_v)
            return carry

        lax.fori_loop(0, nch // 2, pair, 0)
        drain_out(acca_v)
        drain_out(accb_v)

    f = pl.kernel(
        body,
        out_type=jax.ShapeDtypeStruct((s, h), jnp.float32),
        mesh=mesh,
        scratch_types=[
            pltpu.VMEM((tpw,), jnp.int32), pltpu.VMEM((tpw,), jnp.int32),
            pltpu.VMEM((tpw,), jnp.float32), pltpu.VMEM((tpw,), jnp.float32),
            pltpu.VMEM((_COMB_CH, h), jnp.float32),
            pltpu.VMEM((_COMB_CH, h), jnp.float32),
            pltpu.VMEM((_COMB_CH, h), jnp.float32),
            pltpu.VMEM((_COMB_CH, h), jnp.float32),
            pltpu.VMEM((_COMB_CH, h), jnp.float32),
            pltpu.VMEM((_COMB_CH, h), jnp.float32),
            pltpu.SemaphoreType.DMA, pltpu.SemaphoreType.DMA,
            pltpu.SemaphoreType.DMA,
        ],
    )
    return f(eo, s1, s2, w1, w2)


def kernel(inputs, gate_weight, wi, wo):
    orig_shape = inputs.shape
    h = inputs.shape[-1]
    tokens = inputs.reshape(-1, h)
    s = tokens.shape[0]
    e = gate_weight.shape[0]
    cap = math.floor(2 * 1.25 * s / e)
    cap += cap % 2
    cap = max(cap, 4)
    n_disp_rows = e * cap + _ROW_BLK

    t1, t2, r1, c2, p1, p2, tot = _routing_call(tokens, gate_weight, cap)
    t1, t2, r1, c2, p1, p2 = (a.reshape(s) for a in (t1, t2, r1, c2, p1, p2))
    tot = tot.reshape(2 * _LANES)

    disp, s1, s2, w1, w2 = _dispatch_call(
        t1, t2, r1, c2, p1, p2, tot, tokens, cap, n_disp_rows)
    eo = _mlp_call(disp, wi, wo, cap)
    out = _combine_call(eo, s1, s2, w1, w2, (s, h))
    return out.reshape(orig_shape)
